# baseline (speedup 1.0000x reference)
import functools

import jax
import jax.numpy as jnp
from jax import lax
from jax.experimental import pallas as pl
from jax.experimental.pallas import tpu as pltpu
from jax.experimental.pallas import tpu_sc as plsc

_B, _S, _D, _E = 2, 2048, 768, 8
_N = _B * _S
_P = 2 * _N
_TM = 256
_MAXT = _P // _TM + _E
_R = _MAXT * _TM
_TMAP = 48
_NC, _NS = 2, 16
_NW = _NC * _NS
_TPW = _N // _NW


def _gating_body(x_ref, wg_ref, wn_ref, nc_ref, e1_ref, e2_ref, g1_ref,
                 g2_ref):
    x = x_ref[...]
    h = jnp.dot(x, wg_ref[...], preferred_element_type=jnp.float32)
    h = h + nc_ref[...] + jax.nn.softplus(
        jnp.dot(x, wn_ref[...], preferred_element_type=jnp.float32))
    lane = lax.broadcasted_iota(jnp.int32, h.shape, 1)
    m1 = jnp.max(h, axis=-1, keepdims=True)
    e1 = jnp.min(jnp.where(h == m1, lane, _E), axis=-1, keepdims=True)
    h2 = jnp.where(lane == e1, -jnp.inf, h)
    m2 = jnp.max(h2, axis=-1, keepdims=True)
    e2 = jnp.min(jnp.where(h2 == m2, lane, _E), axis=-1, keepdims=True)
    g1 = 1.0 / (1.0 + jnp.exp(m2 - m1))
    e1_ref[...] = e1
    e2_ref[...] = e2
    g1_ref[...] = g1
    g2_ref[...] = 1.0 - g1


def _b1_body(e1_hbm, e2_hbm, cnt_hbm, ebuf, cbuf):
    w = lax.axis_index("s") * _NC + lax.axis_index("c")
    base = w * _TPW
    iota = lax.iota(jnp.int32, 16)
    for s, src in ((0, e1_hbm), (1, e2_hbm)):
        pltpu.sync_copy(src.at[pl.ds(base, _TPW)], ebuf)
        cvec = jnp.zeros((16,), jnp.int32)
        for c in range(_TPW // 16):
            ev = ebuf[pl.ds(c * 16, 16)]
            for b in range(_E):
                cnt = jnp.sum(jnp.where(ev == b, 1, 0))
                cvec = cvec + cnt * jnp.where(iota == b, 1, 0)
        cbuf[...] = cvec
        pltpu.sync_copy(cbuf, cnt_hbm.at[s * _NW + w])


def _b2_body(e1_hbm, e2_hbm, g1_hbm, g2_hbm, cnt_hbm,
             pos_hbm, perm_hbm, gsort_hbm, te_hbm, tv_hbm,
             ctab, ebuf, gbuf, tidbuf, posbuf, tebuf, tvbuf, sem):
    w = lax.axis_index("s") * _NC + lax.axis_index("c")
    base = w * _TPW
    iota = lax.iota(jnp.int32, 16)
    pltpu.sync_copy(cnt_hbm, ctab)
    rows = [ctab[v] for v in range(2 * _NW)]
    totals = rows[0]
    for v in range(1, 2 * _NW):
        totals = totals + rows[v]
    nt = (totals + (_TM - 1)) >> 8
    ntc = plsc.cumsum(nt)
    base_rows = (ntc - nt) * _TM
    acc0 = base_rows
    acc1 = base_rows
    for v in range(2 * _NW):
        vv = jnp.full((16,), v, jnp.int32)
        acc0 = acc0 + jnp.where(vv < w, rows[v], 0)
        acc1 = acc1 + jnp.where(vv < (_NW + w), rows[v], 0)
    starts = []
    for acc in (acc0, acc1):
        starts.append([jnp.sum(jnp.where(iota == b, acc, 0))
                       for b in range(_E)])
    for s, (esrc, gsrc) in enumerate(((e1_hbm, g1_hbm), (e2_hbm, g2_hbm))):
        pltpu.sync_copy(esrc.at[pl.ds(base, _TPW)], ebuf)
        pltpu.sync_copy(gsrc.at[pl.ds(base, _TPW)], gbuf)
        sb = list(starts[s])
        for c in range(_TPW // 16):
            ev = ebuf[pl.ds(c * 16, 16)]
            tidbuf[pl.ds(c * 16, 16)] = iota + (base + c * 16)
            acc = jnp.zeros((16,), jnp.int32)
            for b in range(_E):
                m = ev == b
                mi = jnp.where(m, 1, 0)
                cum = plsc.cumsum(mi)
                acc = jnp.where(m, sb[b] + cum - 1, acc)
                sb[b] = sb[b] + jnp.sum(mi)
            posbuf[pl.ds(c * 16, 16)] = acc
        pltpu.sync_copy(posbuf, pos_hbm.at[pl.ds(s * _N + base, _TPW)])
        pltpu.async_copy(tidbuf, perm_hbm.at[posbuf], sem).wait()
        pltpu.async_copy(gbuf, gsort_hbm.at[posbuf], sem).wait()

    @pl.when(w == 0)
    def _():
        tot_tiles = jnp.sum(jnp.where(iota == (_E - 1), ntc, 0))
        tb = [jnp.sum(jnp.where(iota == b, ntc - nt, 0)) for b in range(_E)]
        for j in range(_TMAP // 16):
            tidx = iota + j * 16
            te = jnp.zeros((16,), jnp.int32)
            for b in range(1, _E):
                te = te + jnp.where(tidx >= tb[b], 1, 0)
            tv = jnp.where(tidx < tot_tiles, 1, 0)
            tebuf[pl.ds(j * 16, 16)] = te
            tvbuf[pl.ds(j * 16, 16)] = tv
        pltpu.sync_copy(tebuf, te_hbm)
        pltpu.sync_copy(tvbuf, tv_hbm)


_CGRP = 64


def _c_body(perm_hbm, x_hbm, xs_hbm, idxbuf, rowsbuf, sem):
    w = lax.axis_index("s") * _NC + lax.axis_index("c")
    rbase = w * (_R // _NW)
    for c in range(_R // _NW // _CGRP):
        off = rbase + c * _CGRP
        pltpu.sync_copy(perm_hbm.at[pl.ds(off, _CGRP)], idxbuf)
        for j in range(_CGRP // 16):
            v = idxbuf[pl.ds(j * 16, 16)]
            idxbuf[pl.ds(j * 16, 16)] = jnp.minimum(
                jnp.maximum(v, 0), _N - 1)
        pltpu.async_copy(x_hbm.at[idxbuf], rowsbuf, sem).wait()
        pltpu.sync_copy(rowsbuf, xs_hbm.at[pl.ds(off, _CGRP)])


def _ffn_body(te_ref, tv_ref, xs_ref, w1_ref, b1_ref, w2_ref, b2_ref,
              gs_ref, o_ref):
    i = pl.program_id(0)

    @pl.when(tv_ref[i] != 0)
    def _():
        e = te_ref[i]
        x = xs_ref[...]
        h = jnp.dot(x, w1_ref[0], preferred_element_type=jnp.float32)
        h = jnp.maximum(h + b1_ref[pl.ds(e, 1)], 0.0)
        out = jnp.dot(h, w2_ref[0], preferred_element_type=jnp.float32)
        out = out + b2_ref[pl.ds(e, 1)]
        o_ref[...] = out * gs_ref[0]


_EGRP = 32


def _e_body(orow_hbm, pos_hbm, y_hbm, ibuf, b0, b1b, sem):
    w = lax.axis_index("s") * _NC + lax.axis_index("c")
    tbase = w * _TPW
    for c in range(_TPW // _EGRP):
        off = tbase + c * _EGRP
        pltpu.sync_copy(pos_hbm.at[pl.ds(off, _EGRP)], ibuf)
        pltpu.async_copy(orow_hbm.at[ibuf], b0, sem).wait()
        pltpu.sync_copy(pos_hbm.at[pl.ds(_N + off, _EGRP)], ibuf)
        pltpu.async_copy(orow_hbm.at[ibuf], b1b, sem).wait()
        for r in range(_EGRP):
            def add_col(i, _, r=r):
                sl = pl.ds(i * 16, 16)
                b0[r, sl] = b0[r, sl] + b1b[r, sl]
                return 0
            lax.fori_loop(0, _D // 16, add_col, 0)
        pltpu.sync_copy(b0, y_hbm.at[pl.ds(off, _EGRP)])


@jax.jit
def _moe(x, W_g, W_noise, W1, b1, W2, b2):
    xf = x.reshape(_N, _D)
    nconst = jax.random.normal(jax.random.key(42), (_B, _S, _E),
                               dtype=jnp.float32).reshape(_N, _E)

    e1, e2, g1, g2 = pl.pallas_call(
        _gating_body,
        out_shape=[
            jax.ShapeDtypeStruct((_N, 1), jnp.int32),
            jax.ShapeDtypeStruct((_N, 1), jnp.int32),
            jax.ShapeDtypeStruct((_N, 1), jnp.float32),
            jax.ShapeDtypeStruct((_N, 1), jnp.float32),
        ],
    )(xf, W_g, W_noise, nconst)
    e1, e2 = e1.reshape(_N), e2.reshape(_N)
    g1, g2 = g1.reshape(_N), g2.reshape(_N)

    mesh = plsc.VectorSubcoreMesh(core_axis_name="c", subcore_axis_name="s")
    _SC_PARAMS = pltpu.CompilerParams(needs_layout_passes=False)

    counts = pl.kernel(
        _b1_body,
        out_type=jax.ShapeDtypeStruct((2 * _NW, 16), jnp.int32),
        mesh=mesh,
        compiler_params=_SC_PARAMS,
        scratch_types=[
            pltpu.VMEM((_TPW,), jnp.int32),
            pltpu.VMEM((16,), jnp.int32),
        ],
    )(e1, e2)

    pos, perm, gsort, te, tv = pl.kernel(
        _b2_body,
        out_type=[
            jax.ShapeDtypeStruct((_P,), jnp.int32),
            jax.ShapeDtypeStruct((_R,), jnp.int32),
            jax.ShapeDtypeStruct((_R,), jnp.float32),
            jax.ShapeDtypeStruct((_TMAP,), jnp.int32),
            jax.ShapeDtypeStruct((_TMAP,), jnp.int32),
        ],
        mesh=mesh,
        compiler_params=_SC_PARAMS,
        scratch_types=[
            pltpu.VMEM((2 * _NW, 16), jnp.int32),
            pltpu.VMEM((_TPW,), jnp.int32),
            pltpu.VMEM((_TPW,), jnp.float32),
            pltpu.VMEM((_TPW,), jnp.int32),
            pltpu.VMEM((_TPW,), jnp.int32),
            pltpu.VMEM((_TMAP,), jnp.int32),
            pltpu.VMEM((_TMAP,), jnp.int32),
            pltpu.SemaphoreType.DMA,
        ],
    )(e1, e2, g1, g2, counts)

    x_sorted = pl.kernel(
        _c_body,
        out_type=jax.ShapeDtypeStruct((_R, _D), jnp.float32),
        mesh=mesh,
        compiler_params=_SC_PARAMS,
        scratch_types=[
            pltpu.VMEM((_CGRP,), jnp.int32),
            pltpu.VMEM((_CGRP, _D), jnp.float32),
            pltpu.SemaphoreType.DMA,
        ],
    )(perm, xf)

    gsr = gsort.reshape(_MAXT, _TM, 1)
    out_rows = pl.pallas_call(
        _ffn_body,
        grid_spec=pltpu.PrefetchScalarGridSpec(
            num_scalar_prefetch=2,
            grid=(_MAXT,),
            in_specs=[
                pl.BlockSpec((_TM, _D), lambda i, te, tv: (i, 0)),
                pl.BlockSpec((1, _D, _D), lambda i, te, tv: (te[i], 0, 0)),
                pl.BlockSpec((_E, _D), lambda i, te, tv: (0, 0)),
                pl.BlockSpec((1, _D, _D), lambda i, te, tv: (te[i], 0, 0)),
                pl.BlockSpec((_E, _D), lambda i, te, tv: (0, 0)),
                pl.BlockSpec((1, _TM, 1), lambda i, te, tv: (i, 0, 0)),
            ],
            out_specs=pl.BlockSpec((_TM, _D), lambda i, te, tv: (i, 0)),
        ),
        out_shape=jax.ShapeDtypeStruct((_R, _D), jnp.float32),
    )(te, tv, x_sorted, W1, b1, W2, b2, gsr)

    y = pl.kernel(
        _e_body,
        out_type=jax.ShapeDtypeStruct((_N, _D), jnp.float32),
        mesh=mesh,
        compiler_params=_SC_PARAMS,
        scratch_types=[
            pltpu.VMEM((_EGRP,), jnp.int32),
            pltpu.VMEM((_EGRP, _D), jnp.float32),
            pltpu.VMEM((_EGRP, _D), jnp.float32),
            pltpu.SemaphoreType.DMA,
        ],
    )(out_rows, pos)

    return y.reshape(_B, _S, _D)


def kernel(x, W_g, W_noise, W1, b1, W2, b2, k):
    return _moe(x, W_g, W_noise, W1, b1, W2, b2)


# speedup vs baseline: 1.5374x; 1.5374x over previous
import jax
import jax.numpy as jnp
from jax import lax
from jax.experimental import pallas as pl
from jax.experimental.pallas import tpu as pltpu
from jax.experimental.pallas import tpu_sc as plsc

_B, _S, _D, _E = 2, 2048, 768, 8
_N = _B * _S
_P = 2 * _N
_TM = 256
_MAXT = _P // _TM + _E
_R = _MAXT * _TM
_TMAP = 48
_NC, _NS = 2, 16
_NW = _NC * _NS
_TPW = _N // _NW
_HC = 64


def _gating_body(x_ref, wg_ref, wn_ref, nc_ref, e1_ref, e2_ref, g1_ref,
                 g2_ref):
    x = x_ref[...]
    h = jnp.dot(x, wg_ref[...], preferred_element_type=jnp.float32)
    h = h + nc_ref[...] + jax.nn.softplus(
        jnp.dot(x, wn_ref[...], preferred_element_type=jnp.float32))
    lane = lax.broadcasted_iota(jnp.int32, h.shape, 1)
    m1 = jnp.max(h, axis=-1, keepdims=True)
    e1 = jnp.min(jnp.where(h == m1, lane, _E), axis=-1, keepdims=True)
    h2 = jnp.where(lane == e1, -jnp.inf, h)
    m2 = jnp.max(h2, axis=-1, keepdims=True)
    e2 = jnp.min(jnp.where(h2 == m2, lane, _E), axis=-1, keepdims=True)
    g1 = 1.0 / (1.0 + jnp.exp(m2 - m1))
    e1_ref[...] = e1
    e2_ref[...] = e2
    g1_ref[...] = g1
    g2_ref[...] = 1.0 - g1


def _b1_body(e1_hbm, e2_hbm, cnt_hbm, ebuf, cbuf):
    w = lax.axis_index("s") * _NC + lax.axis_index("c")
    base = w * _TPW
    iota = lax.iota(jnp.int32, 16)
    for s, src in ((0, e1_hbm), (1, e2_hbm)):
        pltpu.sync_copy(src.at[pl.ds(base, _TPW)], ebuf)

        def hist(c, cvec):
            ev = ebuf[pl.ds(c * 16, 16)]
            for b in range(_E):
                cnt = jnp.sum(jnp.where(ev == b, 1, 0))
                cvec = cvec + cnt * jnp.where(iota == b, 1, 0)
            return cvec

        cbuf[...] = lax.fori_loop(0, _TPW // 16, hist,
                                  jnp.zeros((16,), jnp.int32))
        pltpu.sync_copy(cbuf, cnt_hbm.at[s * _NW + w])


def _b2_body(e1_hbm, e2_hbm, g1_hbm, g2_hbm, cnt_hbm, x_hbm,
             pos_hbm, gsort_hbm, xs_hbm, te_hbm, tv_hbm,
             ctab, ebuf, gbuf, posbuf, pbufs, xrows, tebuf, tvbuf, sem,
             sem2):
    w = lax.axis_index("s") * _NC + lax.axis_index("c")
    base = w * _TPW
    iota = lax.iota(jnp.int32, 16)
    pltpu.sync_copy(cnt_hbm, ctab)
    rows = [ctab[v] for v in range(2 * _NW)]
    totals = rows[0]
    for v in range(1, 2 * _NW):
        totals = totals + rows[v]
    nt = (totals + (_TM - 1)) >> 8
    ntc = plsc.cumsum(nt)
    base_rows = (ntc - nt) * _TM
    acc0 = base_rows
    acc1 = base_rows
    for v in range(2 * _NW):
        vv = jnp.full((16,), v, jnp.int32)
        acc0 = acc0 + jnp.where(vv < w, rows[v], 0)
        acc1 = acc1 + jnp.where(vv < (_NW + w), rows[v], 0)
    starts = []
    for acc in (acc0, acc1):
        starts.append([jnp.sum(jnp.where(iota == b, acc, 0))
                       for b in range(_E)])
    for s, (esrc, gsrc) in enumerate(((e1_hbm, g1_hbm), (e2_hbm, g2_hbm))):
        pltpu.sync_copy(esrc.at[pl.ds(base, _TPW)], ebuf)
        pltpu.sync_copy(gsrc.at[pl.ds(base, _TPW)], gbuf)

        def rank(c, sb, s=s):
            ev = ebuf[pl.ds(c * 16, 16)]
            acc = jnp.zeros((16,), jnp.int32)
            nsb = []
            for b in range(_E):
                m = ev == b
                mi = jnp.where(m, 1, 0)
                cum = plsc.cumsum(mi)
                acc = jnp.where(m, sb[b] + cum - 1, acc)
                nsb.append(sb[b] + jnp.sum(mi))
            posbuf[pl.ds(c * 16, 16)] = acc
            return tuple(nsb)

        lax.fori_loop(0, _TPW // 16, rank, tuple(starts[s]))
        for j in range(_TPW // 16):
            half, quart = divmod(j, (_HC // 16))
            pbufs[2 * s + half][pl.ds(quart * 16, 16)] = \
                posbuf[pl.ds(j * 16, 16)]
        pltpu.sync_copy(pbufs[2 * s], pos_hbm.at[pl.ds(s * _N + base, _HC)])
        pltpu.sync_copy(pbufs[2 * s + 1],
                        pos_hbm.at[pl.ds(s * _N + base + _HC, _HC)])
        d1 = pltpu.async_copy(gbuf.at[pl.ds(0, _HC)],
                              gsort_hbm.at[pbufs[2 * s]], sem2)
        d2 = pltpu.async_copy(gbuf.at[pl.ds(_HC, _HC)],
                              gsort_hbm.at[pbufs[2 * s + 1]], sem2)
        d1.wait()
        d2.wait()
    for half in range(_TPW // _HC):
        pltpu.sync_copy(x_hbm.at[pl.ds(base + half * _HC, _HC)], xrows)
        d1 = pltpu.async_copy(xrows, xs_hbm.at[pbufs[half]], sem)
        d2 = pltpu.async_copy(xrows, xs_hbm.at[pbufs[2 + half]], sem)
        d1.wait()
        d2.wait()

    @pl.when(w == 0)
    def _():
        tot_tiles = jnp.sum(jnp.where(iota == (_E - 1), ntc, 0))
        tb = [jnp.sum(jnp.where(iota == b, ntc - nt, 0)) for b in range(_E)]
        for j in range(_TMAP // 16):
            tidx = iota + j * 16
            te = jnp.zeros((16,), jnp.int32)
            for b in range(1, _E):
                te = te + jnp.where(tidx >= tb[b], 1, 0)
            tv = jnp.where(tidx < tot_tiles, 1, 0)
            tebuf[pl.ds(j * 16, 16)] = te
            tvbuf[pl.ds(j * 16, 16)] = tv
        pltpu.sync_copy(tebuf, te_hbm)
        pltpu.sync_copy(tvbuf, tv_hbm)


def _ffn_body(te_ref, tv_ref, xs_ref, w1_ref, b1_ref, w2_ref, b2_ref,
              gs_ref, o_ref):
    i = pl.program_id(0)

    @pl.when(tv_ref[i] != 0)
    def _():
        e = te_ref[i]
        x = xs_ref[...]
        h = jnp.dot(x, w1_ref[0], preferred_element_type=jnp.float32)
        h = jnp.maximum(h + b1_ref[pl.ds(e, 1)], 0.0)
        out = jnp.dot(h, w2_ref[0], preferred_element_type=jnp.float32)
        out = out + b2_ref[pl.ds(e, 1)]
        o_ref[...] = out * gs_ref[0]


_EGRP = 16


def _add_rows(dst, src):
    def row(r, _):
        def body(i, _):
            for u in range(8):
                sl = pl.ds(i * 128 + u * 16, 16)
                dst[r, sl] = dst[r, sl] + src[r, sl]
            return 0
        return lax.fori_loop(0, _D // 128, body, 0)
    lax.fori_loop(0, _EGRP, row, 0)


def _e_body(orow_hbm, pos_hbm, y_hbm, ibufs, bufs, sem, sem2):
    w = lax.axis_index("s") * _NC + lax.axis_index("c")
    tbase = w * _TPW
    nch = _TPW // _EGRP
    sems = (sem, sem2)

    def fire(c, par):
        off = tbase + c * _EGRP
        i0, i1 = ibufs[2 * par], ibufs[2 * par + 1]
        pltpu.sync_copy(pos_hbm.at[pl.ds(off, _EGRP)], i0)
        d1 = pltpu.async_copy(orow_hbm.at[i0], bufs[2 * par], sems[par])
        pltpu.sync_copy(pos_hbm.at[pl.ds(_N + off, _EGRP)], i1)
        d2 = pltpu.async_copy(orow_hbm.at[i1], bufs[2 * par + 1], sems[par])
        return d1, d2

    pend = fire(0, 0)
    for c in range(nch):
        par = c % 2
        nxt = fire(c + 1, 1 - par) if c + 1 < nch else None
        pend[0].wait()
        pend[1].wait()
        _add_rows(bufs[2 * par], bufs[2 * par + 1])
        pltpu.sync_copy(bufs[2 * par],
                        y_hbm.at[pl.ds(tbase + c * _EGRP, _EGRP)])
        pend = nxt


@jax.jit
def _moe(x, W_g, W_noise, W1, b1, W2, b2):
    xf = x.reshape(_N, _D)
    nconst = jax.random.normal(jax.random.key(42), (_B, _S, _E),
                               dtype=jnp.float32).reshape(_N, _E)

    e1, e2, g1, g2 = pl.pallas_call(
        _gating_body,
        out_shape=[
            jax.ShapeDtypeStruct((_N, 1), jnp.int32),
            jax.ShapeDtypeStruct((_N, 1), jnp.int32),
            jax.ShapeDtypeStruct((_N, 1), jnp.float32),
            jax.ShapeDtypeStruct((_N, 1), jnp.float32),
        ],
    )(xf, W_g, W_noise, nconst)
    e1, e2 = e1.reshape(_N), e2.reshape(_N)
    g1, g2 = g1.reshape(_N), g2.reshape(_N)

    mesh = plsc.VectorSubcoreMesh(core_axis_name="c", subcore_axis_name="s")
    scp = pltpu.CompilerParams(needs_layout_passes=False)

    counts = pl.kernel(
        _b1_body,
        out_type=jax.ShapeDtypeStruct((2 * _NW, 16), jnp.int32),
        mesh=mesh,
        compiler_params=scp,
        scratch_types=[
            pltpu.VMEM((_TPW,), jnp.int32),
            pltpu.VMEM((16,), jnp.int32),
        ],
    )(e1, e2)

    pos, gsort, x_sorted, te, tv = pl.kernel(
        _b2_body,
        out_type=[
            jax.ShapeDtypeStruct((_P,), jnp.int32),
            jax.ShapeDtypeStruct((_R,), jnp.float32),
            jax.ShapeDtypeStruct((_R, _D), jnp.float32),
            jax.ShapeDtypeStruct((_TMAP,), jnp.int32),
            jax.ShapeDtypeStruct((_TMAP,), jnp.int32),
        ],
        mesh=mesh,
        compiler_params=scp,
        scratch_types=[
            pltpu.VMEM((2 * _NW, 16), jnp.int32),
            pltpu.VMEM((_TPW,), jnp.int32),
            pltpu.VMEM((_TPW,), jnp.float32),
            pltpu.VMEM((_TPW,), jnp.int32),
            [pltpu.VMEM((_HC,), jnp.int32) for _ in range(4)],
            pltpu.VMEM((_HC, _D), jnp.float32),
            pltpu.VMEM((_TMAP,), jnp.int32),
            pltpu.VMEM((_TMAP,), jnp.int32),
            pltpu.SemaphoreType.DMA,
            pltpu.SemaphoreType.DMA,
        ],
    )(e1, e2, g1, g2, counts, xf)

    gsr = gsort.reshape(_MAXT, _TM, 1)
    out_rows = pl.pallas_call(
        _ffn_body,
        grid_spec=pltpu.PrefetchScalarGridSpec(
            num_scalar_prefetch=2,
            grid=(_MAXT,),
            in_specs=[
                pl.BlockSpec((_TM, _D), lambda i, te, tv: (i, 0)),
                pl.BlockSpec((1, _D, _D), lambda i, te, tv: (te[i], 0, 0)),
                pl.BlockSpec((_E, _D), lambda i, te, tv: (0, 0)),
                pl.BlockSpec((1, _D, _D), lambda i, te, tv: (te[i], 0, 0)),
                pl.BlockSpec((_E, _D), lambda i, te, tv: (0, 0)),
                pl.BlockSpec((1, _TM, 1), lambda i, te, tv: (i, 0, 0)),
            ],
            out_specs=pl.BlockSpec((_TM, _D), lambda i, te, tv: (i, 0)),
        ),
        out_shape=jax.ShapeDtypeStruct((_R, _D), jnp.float32),
    )(te, tv, x_sorted, W1, b1, W2, b2, gsr)

    y = pl.kernel(
        _e_body,
        out_type=jax.ShapeDtypeStruct((_N, _D), jnp.float32),
        mesh=mesh,
        compiler_params=scp,
        scratch_types=[
            [pltpu.VMEM((_EGRP,), jnp.int32) for _ in range(4)],
            [pltpu.VMEM((_EGRP, _D), jnp.float32) for _ in range(4)],
            pltpu.SemaphoreType.DMA,
            pltpu.SemaphoreType.DMA,
        ],
    )(out_rows, pos)

    return y.reshape(_B, _S, _D)


def kernel(x, W_g, W_noise, W1, b1, W2, b2, k):
    return _moe(x, W_g, W_noise, W1, b1, W2, b2)


# speedup vs baseline: 1.7537x; 1.1407x over previous
import jax
import jax.numpy as jnp
from jax import lax
from jax.experimental import pallas as pl
from jax.experimental.pallas import tpu as pltpu
from jax.experimental.pallas import tpu_sc as plsc

_B, _S, _D, _E = 2, 2048, 768, 8
_N = _B * _S
_P = 2 * _N
_TM = 256
_MAXT = _P // _TM + _E
_R = _MAXT * _TM
_TMAP = 48
_NC, _NS = 2, 16
_NW = _NC * _NS
_TPW = _N // _NW
_TG = 1024
_WPT = _TG // _TPW
_XC = 32


def _gating_body(x_ref, wg_ref, wn_ref, nc_ref, e1_ref, e2_ref, g1_ref,
                 g2_ref, c0_ref, c1_ref):
    x = x_ref[...]
    h = jnp.dot(x, wg_ref[...], preferred_element_type=jnp.float32)
    h = h + nc_ref[...] + jax.nn.softplus(
        jnp.dot(x, wn_ref[...], preferred_element_type=jnp.float32))
    lane = lax.broadcasted_iota(jnp.int32, h.shape, 1)
    m1 = jnp.max(h, axis=-1, keepdims=True)
    e1 = jnp.min(jnp.where(h == m1, lane, _E), axis=-1, keepdims=True)
    h2 = jnp.where(lane == e1, -jnp.inf, h)
    m2 = jnp.max(h2, axis=-1, keepdims=True)
    e2 = jnp.min(jnp.where(h2 == m2, lane, _E), axis=-1, keepdims=True)
    g1 = 1.0 / (1.0 + jnp.exp(m2 - m1))
    e1_ref[...] = e1
    e2_ref[...] = e2
    g1_ref[...] = g1
    g2_ref[...] = 1.0 - g1
    rowi = lax.broadcasted_iota(jnp.int32, (_WPT, _TG), 0)
    tokw = lax.broadcasted_iota(jnp.int32, (_WPT, _TG), 1) // _TPW
    sel = jnp.where(rowi == tokw, 1.0, 0.0)
    lane16 = lax.broadcasted_iota(jnp.int32, (_TG, 16), 1)
    oh1 = jnp.where(lane16 == e1, 1.0, 0.0)
    oh2 = jnp.where(lane16 == e2, 1.0, 0.0)
    c0_ref[...] = jnp.dot(sel, oh1,
                          preferred_element_type=jnp.float32).astype(
                              jnp.int32)
    c1_ref[...] = jnp.dot(sel, oh2,
                          preferred_element_type=jnp.float32).astype(
                              jnp.int32)


def _b2_body(e1_hbm, e2_hbm, g1_hbm, g2_hbm, c0_hbm, c1_hbm, x_hbm,
             pos_hbm, gsort_hbm, xs_hbm, te_hbm, tv_hbm,
             ctab, ebuf, gbufs, posbuf, pbufs, xbufs, tebuf, tvbuf, seml,
             sems, semg):
    w = lax.axis_index("s") * _NC + lax.axis_index("c")
    base = w * _TPW
    iota = lax.iota(jnp.int32, 16)
    pltpu.sync_copy(c0_hbm, ctab.at[pl.ds(0, _NW)])
    pltpu.sync_copy(c1_hbm, ctab.at[pl.ds(_NW, _NW)])
    rows = [ctab[v] for v in range(2 * _NW)]
    totals = rows[0]
    for v in range(1, 2 * _NW):
        totals = totals + rows[v]
    nt = (totals + (_TM - 1)) >> 8
    ntc = plsc.cumsum(nt)
    base_rows = (ntc - nt) * _TM
    acc0 = base_rows
    acc1 = base_rows
    for v in range(2 * _NW):
        vv = jnp.full((16,), v, jnp.int32)
        acc0 = acc0 + jnp.where(vv < w, rows[v], 0)
        acc1 = acc1 + jnp.where(vv < (_NW + w), rows[v], 0)
    starts = []
    for acc in (acc0, acc1):
        starts.append([jnp.sum(jnp.where(iota == b, acc, 0))
                       for b in range(_E)])
    gds = []
    for s, (esrc, gsrc) in enumerate(((e1_hbm, g1_hbm), (e2_hbm, g2_hbm))):
        pltpu.sync_copy(esrc.at[pl.ds(base, _TPW)], ebuf)
        pltpu.sync_copy(gsrc.at[pl.ds(base, _TPW)], gbufs[s])

        def rank(c, sb, s=s):
            ev = ebuf[pl.ds(c * 16, 16)]
            acc = jnp.zeros((16,), jnp.int32)
            nsb = []
            for b in range(_E):
                m = ev == b
                mi = jnp.where(m, 1, 0)
                cum = plsc.cumsum(mi)
                acc = jnp.where(m, sb[b] + cum - 1, acc)
                nsb.append(sb[b] + jnp.sum(mi))
            posbuf[pl.ds(c * 16, 16)] = acc
            return tuple(nsb)

        lax.fori_loop(0, _TPW // 16, rank, tuple(starts[s]))
        for j in range(_TPW // 16):
            sub, half = divmod(j, 2)
            pbufs[4 * s + sub][pl.ds(half * 16, 16)] = \
                posbuf[pl.ds(j * 16, 16)]
        pltpu.sync_copy(posbuf, pos_hbm.at[pl.ds(s * _N + base, _TPW)])
        for sub in range(_TPW // _XC):
            gds.append(pltpu.async_copy(
                gbufs[s].at[pl.ds(sub * _XC, _XC)],
                gsort_hbm.at[pbufs[4 * s + sub]], semg))
    nsub = _TPW // _XC
    dload = [pltpu.async_copy(x_hbm.at[pl.ds(base + k * _XC, _XC)],
                              xbufs[k % 2], seml) for k in range(2)]
    dscat = []
    for k in range(nsub):
        dload[k].wait()
        d1 = pltpu.async_copy(xbufs[k % 2], xs_hbm.at[pbufs[k]], sems)
        d2 = pltpu.async_copy(xbufs[k % 2], xs_hbm.at[pbufs[4 + k]], sems)
        dscat.append((d1, d2))
        if k + 2 < nsub:
            d1.wait()
            d2.wait()
            dscat[k] = None
            dload.append(pltpu.async_copy(
                x_hbm.at[pl.ds(base + (k + 2) * _XC, _XC)],
                xbufs[k % 2], seml))
    for d in dscat:
        if d is not None:
            d[0].wait()
            d[1].wait()
    for d in gds:
        d.wait()

    @pl.when(w == 0)
    def _():
        tot_tiles = jnp.sum(jnp.where(iota == (_E - 1), ntc, 0))
        tb = [jnp.sum(jnp.where(iota == b, ntc - nt, 0)) for b in range(_E)]
        for j in range(_TMAP // 16):
            tidx = iota + j * 16
            te = jnp.zeros((16,), jnp.int32)
            for b in range(1, _E):
                te = te + jnp.where(tidx >= tb[b], 1, 0)
            tv = jnp.where(tidx < tot_tiles, 1, 0)
            tebuf[pl.ds(j * 16, 16)] = te
            tvbuf[pl.ds(j * 16, 16)] = tv
        pltpu.sync_copy(tebuf, te_hbm)
        pltpu.sync_copy(tvbuf, tv_hbm)


def _ffn_body(te_ref, tv_ref, xs_ref, w1_ref, b1_ref, w2_ref, b2_ref,
              gs_ref, o_ref):
    i = pl.program_id(0)

    @pl.when(tv_ref[i] != 0)
    def _():
        e = te_ref[i]
        x = xs_ref[...]
        h = jnp.dot(x, w1_ref[0], preferred_element_type=jnp.float32)
        h = jnp.maximum(h + b1_ref[pl.ds(e, 1)], 0.0)
        out = jnp.dot(h, w2_ref[0], preferred_element_type=jnp.float32)
        out = out + b2_ref[pl.ds(e, 1)]
        o_ref[...] = out * gs_ref[0]


_EGRP = 16


def _add_rows(dst, src):
    def row(r, _):
        for u in range(_D // 16):
            sl = pl.ds(u * 16, 16)
            dst[r, sl] = dst[r, sl] + src[r, sl]
        return 0
    lax.fori_loop(0, _EGRP, row, 0)


def _e_body(orow_hbm, pos_hbm, y_hbm, ibufs, bufs, sem, sem2):
    w = lax.axis_index("s") * _NC + lax.axis_index("c")
    tbase = w * _TPW
    nch = _TPW // _EGRP
    sems = (sem, sem2)

    def fire(c, par):
        off = tbase + c * _EGRP
        i0, i1 = ibufs[2 * par], ibufs[2 * par + 1]
        pltpu.sync_copy(pos_hbm.at[pl.ds(off, _EGRP)], i0)
        d1 = pltpu.async_copy(orow_hbm.at[i0], bufs[2 * par], sems[par])
        pltpu.sync_copy(pos_hbm.at[pl.ds(_N + off, _EGRP)], i1)
        d2 = pltpu.async_copy(orow_hbm.at[i1], bufs[2 * par + 1], sems[par])
        return d1, d2

    pend = fire(0, 0)
    for c in range(nch):
        par = c % 2
        nxt = fire(c + 1, 1 - par) if c + 1 < nch else None
        pend[0].wait()
        pend[1].wait()
        _add_rows(bufs[2 * par], bufs[2 * par + 1])
        pltpu.sync_copy(bufs[2 * par],
                        y_hbm.at[pl.ds(tbase + c * _EGRP, _EGRP)])
        pend = nxt


@jax.jit
def _moe(x, W_g, W_noise, W1, b1, W2, b2):
    xf = x.reshape(_N, _D)
    nconst = jax.random.normal(jax.random.key(42), (_B, _S, _E),
                               dtype=jnp.float32).reshape(_N, _E)

    e1, e2, g1, g2, c0, c1 = pl.pallas_call(
        _gating_body,
        grid=(_N // _TG,),
        in_specs=[
            pl.BlockSpec((_TG, _D), lambda t: (t, 0)),
            pl.BlockSpec((_D, _E), lambda t: (0, 0)),
            pl.BlockSpec((_D, _E), lambda t: (0, 0)),
            pl.BlockSpec((_TG, _E), lambda t: (t, 0)),
        ],
        out_specs=[
            pl.BlockSpec((_TG, 1), lambda t: (t, 0)),
            pl.BlockSpec((_TG, 1), lambda t: (t, 0)),
            pl.BlockSpec((_TG, 1), lambda t: (t, 0)),
            pl.BlockSpec((_TG, 1), lambda t: (t, 0)),
            pl.BlockSpec((_WPT, 16), lambda t: (t, 0)),
            pl.BlockSpec((_WPT, 16), lambda t: (t, 0)),
        ],
        out_shape=[
            jax.ShapeDtypeStruct((_N, 1), jnp.int32),
            jax.ShapeDtypeStruct((_N, 1), jnp.int32),
            jax.ShapeDtypeStruct((_N, 1), jnp.float32),
            jax.ShapeDtypeStruct((_N, 1), jnp.float32),
            jax.ShapeDtypeStruct((_NW, 16), jnp.int32),
            jax.ShapeDtypeStruct((_NW, 16), jnp.int32),
        ],
    )(xf, W_g, W_noise, nconst)
    e1, e2 = e1.reshape(_N), e2.reshape(_N)
    g1, g2 = g1.reshape(_N), g2.reshape(_N)

    mesh = plsc.VectorSubcoreMesh(core_axis_name="c", subcore_axis_name="s")
    scp = pltpu.CompilerParams(needs_layout_passes=False)

    pos, gsort, x_sorted, te, tv = pl.kernel(
        _b2_body,
        out_type=[
            jax.ShapeDtypeStruct((_P,), jnp.int32),
            jax.ShapeDtypeStruct((_R,), jnp.float32),
            jax.ShapeDtypeStruct((_R, _D), jnp.float32),
            jax.ShapeDtypeStruct((_TMAP,), jnp.int32),
            jax.ShapeDtypeStruct((_TMAP,), jnp.int32),
        ],
        mesh=mesh,
        compiler_params=scp,
        scratch_types=[
            pltpu.VMEM((2 * _NW, 16), jnp.int32),
            pltpu.VMEM((_TPW,), jnp.int32),
            [pltpu.VMEM((_TPW,), jnp.float32) for _ in range(2)],
            pltpu.VMEM((_TPW,), jnp.int32),
            [pltpu.VMEM((_XC,), jnp.int32) for _ in range(8)],
            [pltpu.VMEM((_XC, _D), jnp.float32) for _ in range(2)],
            pltpu.VMEM((_TMAP,), jnp.int32),
            pltpu.VMEM((_TMAP,), jnp.int32),
            pltpu.SemaphoreType.DMA,
            pltpu.SemaphoreType.DMA,
            pltpu.SemaphoreType.DMA,
        ],
    )(e1, e2, g1, g2, c0, c1, xf)

    gsr = gsort.reshape(_MAXT, _TM, 1)
    out_rows = pl.pallas_call(
        _ffn_body,
        grid_spec=pltpu.PrefetchScalarGridSpec(
            num_scalar_prefetch=2,
            grid=(_MAXT,),
            in_specs=[
                pl.BlockSpec((_TM, _D), lambda i, te, tv: (i, 0)),
                pl.BlockSpec((1, _D, _D), lambda i, te, tv: (te[i], 0, 0)),
                pl.BlockSpec((_E, _D), lambda i, te, tv: (0, 0)),
                pl.BlockSpec((1, _D, _D), lambda i, te, tv: (te[i], 0, 0)),
                pl.BlockSpec((_E, _D), lambda i, te, tv: (0, 0)),
                pl.BlockSpec((1, _TM, 1), lambda i, te, tv: (i, 0, 0)),
            ],
            out_specs=pl.BlockSpec((_TM, _D), lambda i, te, tv: (i, 0)),
        ),
        out_shape=jax.ShapeDtypeStruct((_R, _D), jnp.float32),
    )(te, tv, x_sorted, W1, b1, W2, b2, gsr)

    y = pl.kernel(
        _e_body,
        out_type=jax.ShapeDtypeStruct((_N, _D), jnp.float32),
        mesh=mesh,
        compiler_params=scp,
        scratch_types=[
            [pltpu.VMEM((_EGRP,), jnp.int32) for _ in range(4)],
            [pltpu.VMEM((_EGRP, _D), jnp.float32) for _ in range(4)],
            pltpu.SemaphoreType.DMA,
            pltpu.SemaphoreType.DMA,
        ],
    )(out_rows, pos)

    return y.reshape(_B, _S, _D)


def kernel(x, W_g, W_noise, W1, b1, W2, b2, k):
    return _moe(x, W_g, W_noise, W1, b1, W2, b2)


# speedup vs baseline: 2.3217x; 1.3239x over previous
import jax
import jax.numpy as jnp
from jax import lax
from jax.experimental import pallas as pl
from jax.experimental.pallas import tpu as pltpu
from jax.experimental.pallas import tpu_sc as plsc

_B, _S, _D, _E = 2, 2048, 768, 8
_N = _B * _S
_P = 2 * _N
_TM = 256
_MAXT = _P // _TM + _E
_R = _MAXT * _TM
_TMAP = 48
_NC, _NS = 2, 16
_NW = _NC * _NS
_TPW = _N // _NW
_TG = 1024
_WPT = _TG // _TPW
_XC = 32


def _gating_body(x_ref, wg_ref, wn_ref, nc_ref, e1_ref, e2_ref, g1_ref,
                 g2_ref, c0_ref, c1_ref):
    x = x_ref[...]
    h = jnp.dot(x, wg_ref[...], preferred_element_type=jnp.float32)
    h = h + nc_ref[...] + jax.nn.softplus(
        jnp.dot(x, wn_ref[...], preferred_element_type=jnp.float32))
    lane = lax.broadcasted_iota(jnp.int32, h.shape, 1)
    m1 = jnp.max(h, axis=-1, keepdims=True)
    e1 = jnp.min(jnp.where(h == m1, lane, _E), axis=-1, keepdims=True)
    h2 = jnp.where(lane == e1, -jnp.inf, h)
    m2 = jnp.max(h2, axis=-1, keepdims=True)
    e2 = jnp.min(jnp.where(h2 == m2, lane, _E), axis=-1, keepdims=True)
    g1 = 1.0 / (1.0 + jnp.exp(m2 - m1))
    e1_ref[...] = e1
    e2_ref[...] = e2
    g1_ref[...] = g1
    g2_ref[...] = 1.0 - g1
    rowi = lax.broadcasted_iota(jnp.int32, (_WPT, _TG), 0)
    tokw = lax.broadcasted_iota(jnp.int32, (_WPT, _TG), 1) // _TPW
    sel = jnp.where(rowi == tokw, 1.0, 0.0)
    lane16 = lax.broadcasted_iota(jnp.int32, (_TG, 16), 1)
    oh1 = jnp.where(lane16 == e1, 1.0, 0.0)
    oh2 = jnp.where(lane16 == e2, 1.0, 0.0)
    c0_ref[...] = jnp.dot(sel, oh1,
                          preferred_element_type=jnp.float32).astype(
                              jnp.int32)
    c1_ref[...] = jnp.dot(sel, oh2,
                          preferred_element_type=jnp.float32).astype(
                              jnp.int32)


def _b2_body(e1_hbm, e2_hbm, c0_hbm, c1_hbm, x_hbm,
             pos_hbm, xs_hbm, te_hbm, tv_hbm,
             ctab, ebuf, posbuf, pbufs, xbufs, tebuf, tvbuf, seml, sems):
    w = lax.axis_index("s") * _NC + lax.axis_index("c")
    base = w * _TPW
    iota = lax.iota(jnp.int32, 16)
    dload = [pltpu.async_copy(x_hbm.at[pl.ds(base + k * _XC, _XC)],
                              xbufs[k], seml) for k in range(3)]
    pltpu.sync_copy(c0_hbm, ctab.at[pl.ds(0, _NW)])
    pltpu.sync_copy(c1_hbm, ctab.at[pl.ds(_NW, _NW)])
    rows = [ctab[v] for v in range(2 * _NW)]
    totals = rows[0]
    for v in range(1, 2 * _NW):
        totals = totals + rows[v]
    nt = (totals + (_TM - 1)) >> 8
    ntc = plsc.cumsum(nt)
    base_rows = (ntc - nt) * _TM
    acc0 = base_rows
    acc1 = base_rows
    for v in range(2 * _NW):
        vv = jnp.full((16,), v, jnp.int32)
        acc0 = acc0 + jnp.where(vv < w, rows[v], 0)
        acc1 = acc1 + jnp.where(vv < (_NW + w), rows[v], 0)
    starts = []
    for acc in (acc0, acc1):
        starts.append([jnp.sum(jnp.where(iota == b, acc, 0))
                       for b in range(_E)])
    for s, esrc in enumerate((e1_hbm, e2_hbm)):
        pltpu.sync_copy(esrc.at[pl.ds(base, _TPW)], ebuf)

        def rank(c, sb, s=s):
            ev = ebuf[pl.ds(c * 16, 16)]
            acc = jnp.zeros((16,), jnp.int32)
            nsb = []
            for b in range(_E):
                m = ev == b
                mi = jnp.where(m, 1, 0)
                cum = plsc.cumsum(mi)
                acc = jnp.where(m, sb[b] + cum - 1, acc)
                nsb.append(sb[b] + jnp.sum(mi))
            posbuf[pl.ds(c * 16, 16)] = acc
            return tuple(nsb)

        lax.fori_loop(0, _TPW // 16, rank, tuple(starts[s]))
        for j in range(_TPW // 16):
            sub, half = divmod(j, 2)
            pbufs[4 * s + sub][pl.ds(half * 16, 16)] = \
                posbuf[pl.ds(j * 16, 16)]
        pltpu.sync_copy(posbuf, pos_hbm.at[pl.ds(s * _N + base, _TPW)])
    dscat = []
    for k in range(3):
        dload[k].wait()
        dscat.append(pltpu.async_copy(xbufs[k], xs_hbm.at[pbufs[k]], sems))
        dscat.append(pltpu.async_copy(xbufs[k], xs_hbm.at[pbufs[4 + k]],
                                      sems))
    dscat[0].wait()
    dscat[1].wait()
    pltpu.sync_copy(x_hbm.at[pl.ds(base + 3 * _XC, _XC)], xbufs[0])
    dscat.append(pltpu.async_copy(xbufs[0], xs_hbm.at[pbufs[3]], sems))
    dscat.append(pltpu.async_copy(xbufs[0], xs_hbm.at[pbufs[7]], sems))
    for d in dscat[2:]:
        d.wait()

    @pl.when(w == 0)
    def _():
        tot_tiles = jnp.sum(jnp.where(iota == (_E - 1), ntc, 0))
        tb = [jnp.sum(jnp.where(iota == b, ntc - nt, 0)) for b in range(_E)]
        for j in range(_TMAP // 16):
            tidx = iota + j * 16
            te = jnp.zeros((16,), jnp.int32)
            for b in range(1, _E):
                te = te + jnp.where(tidx >= tb[b], 1, 0)
            tv = jnp.where(tidx < tot_tiles, 1, 0)
            tebuf[pl.ds(j * 16, 16)] = te
            tvbuf[pl.ds(j * 16, 16)] = tv
        pltpu.sync_copy(tebuf, te_hbm)
        pltpu.sync_copy(tvbuf, tv_hbm)


def _ffn_body(te_ref, tv_ref, xs_ref, w1_ref, b1_ref, w2_ref, b2_ref,
              o_ref):
    i = pl.program_id(0)

    @pl.when(tv_ref[i] != 0)
    def _():
        e = te_ref[i]
        x = xs_ref[...]
        h = jnp.dot(x, w1_ref[0], preferred_element_type=jnp.float32)
        h = jnp.maximum(h + b1_ref[pl.ds(e, 1)], 0.0)
        out = jnp.dot(h, w2_ref[0], preferred_element_type=jnp.float32)
        o_ref[...] = out + b2_ref[pl.ds(e, 1)]


_EGRP = 16


def _e_body(orow_hbm, pos_hbm, g1_hbm, g2_hbm, y_hbm, ibufs, gbufs, bufs,
            sem, sem2):
    w = lax.axis_index("s") * _NC + lax.axis_index("c")
    tbase = w * _TPW
    nch = _TPW // _EGRP
    sems = (sem, sem2)
    iota = lax.iota(jnp.int32, 16)

    def fire(c, par):
        off = tbase + c * _EGRP
        i0, i1 = ibufs[2 * par], ibufs[2 * par + 1]
        pltpu.sync_copy(pos_hbm.at[pl.ds(off, _EGRP)], i0)
        d1 = pltpu.async_copy(orow_hbm.at[i0], bufs[2 * par], sems[par])
        pltpu.sync_copy(pos_hbm.at[pl.ds(_N + off, _EGRP)], i1)
        d2 = pltpu.async_copy(orow_hbm.at[i1], bufs[2 * par + 1], sems[par])
        pltpu.sync_copy(g1_hbm.at[pl.ds(off, _EGRP)], gbufs[2 * par])
        pltpu.sync_copy(g2_hbm.at[pl.ds(off, _EGRP)], gbufs[2 * par + 1])
        return d1, d2

    pend = fire(0, 0)
    for c in range(nch):
        par = c % 2
        nxt = fire(c + 1, 1 - par) if c + 1 < nch else None
        pend[0].wait()
        pend[1].wait()
        a, bb = bufs[2 * par], bufs[2 * par + 1]
        gv0 = gbufs[2 * par][...]
        gv1 = gbufs[2 * par + 1][...]

        def row(r, _):
            s0 = jnp.sum(jnp.where(iota == r, gv0, 0.0))
            s1 = jnp.sum(jnp.where(iota == r, gv1, 0.0))
            for u in range(_D // 16):
                sl = pl.ds(u * 16, 16)
                a[r, sl] = s0 * a[r, sl] + s1 * bb[r, sl]
            return 0

        lax.fori_loop(0, _EGRP, row, 0)
        pltpu.sync_copy(a, y_hbm.at[pl.ds(tbase + c * _EGRP, _EGRP)])
        pend = nxt


_NCONST_CACHE = []


def _noise_const():
    if not _NCONST_CACHE:
        _NCONST_CACHE.append(
            jax.random.normal(jax.random.key(42), (_B, _S, _E),
                              dtype=jnp.float32).reshape(_N, _E))
    return _NCONST_CACHE[0]


@jax.jit
def _moe(x, W_g, W_noise, W1, b1, W2, b2):
    xf = x.reshape(_N, _D)
    nconst = _noise_const()

    e1, e2, g1, g2, c0, c1 = pl.pallas_call(
        _gating_body,
        grid=(_N // _TG,),
        in_specs=[
            pl.BlockSpec((_TG, _D), lambda t: (t, 0)),
            pl.BlockSpec((_D, _E), lambda t: (0, 0)),
            pl.BlockSpec((_D, _E), lambda t: (0, 0)),
            pl.BlockSpec((_TG, _E), lambda t: (t, 0)),
        ],
        out_specs=[
            pl.BlockSpec((_TG, 1), lambda t: (t, 0)),
            pl.BlockSpec((_TG, 1), lambda t: (t, 0)),
            pl.BlockSpec((_TG, 1), lambda t: (t, 0)),
            pl.BlockSpec((_TG, 1), lambda t: (t, 0)),
            pl.BlockSpec((_WPT, 16), lambda t: (t, 0)),
            pl.BlockSpec((_WPT, 16), lambda t: (t, 0)),
        ],
        out_shape=[
            jax.ShapeDtypeStruct((_N, 1), jnp.int32),
            jax.ShapeDtypeStruct((_N, 1), jnp.int32),
            jax.ShapeDtypeStruct((_N, 1), jnp.float32),
            jax.ShapeDtypeStruct((_N, 1), jnp.float32),
            jax.ShapeDtypeStruct((_NW, 16), jnp.int32),
            jax.ShapeDtypeStruct((_NW, 16), jnp.int32),
        ],
    )(xf, W_g, W_noise, nconst)
    e1, e2 = e1.reshape(_N), e2.reshape(_N)
    g1, g2 = g1.reshape(_N), g2.reshape(_N)

    mesh = plsc.VectorSubcoreMesh(core_axis_name="c", subcore_axis_name="s")
    scp = pltpu.CompilerParams(needs_layout_passes=False)

    pos, x_sorted, te, tv = pl.kernel(
        _b2_body,
        out_type=[
            jax.ShapeDtypeStruct((_P,), jnp.int32),
            jax.ShapeDtypeStruct((_R, _D), jnp.float32),
            jax.ShapeDtypeStruct((_TMAP,), jnp.int32),
            jax.ShapeDtypeStruct((_TMAP,), jnp.int32),
        ],
        mesh=mesh,
        compiler_params=scp,
        scratch_types=[
            pltpu.VMEM((2 * _NW, 16), jnp.int32),
            pltpu.VMEM((_TPW,), jnp.int32),
            pltpu.VMEM((_TPW,), jnp.int32),
            [pltpu.VMEM((_XC,), jnp.int32) for _ in range(8)],
            [pltpu.VMEM((_XC, _D), jnp.float32) for _ in range(3)],
            pltpu.VMEM((_TMAP,), jnp.int32),
            pltpu.VMEM((_TMAP,), jnp.int32),
            pltpu.SemaphoreType.DMA,
            pltpu.SemaphoreType.DMA,
        ],
    )(e1, e2, c0, c1, xf)

    out_rows = pl.pallas_call(
        _ffn_body,
        grid_spec=pltpu.PrefetchScalarGridSpec(
            num_scalar_prefetch=2,
            grid=(_MAXT,),
            in_specs=[
                pl.BlockSpec((_TM, _D), lambda i, te, tv: (i, 0)),
                pl.BlockSpec((1, _D, _D), lambda i, te, tv: (te[i], 0, 0)),
                pl.BlockSpec((_E, _D), lambda i, te, tv: (0, 0)),
                pl.BlockSpec((1, _D, _D), lambda i, te, tv: (te[i], 0, 0)),
                pl.BlockSpec((_E, _D), lambda i, te, tv: (0, 0)),
            ],
            out_specs=pl.BlockSpec((_TM, _D), lambda i, te, tv: (i, 0)),
        ),
        out_shape=jax.ShapeDtypeStruct((_R, _D), jnp.float32),
    )(te, tv, x_sorted, W1, b1, W2, b2)

    y = pl.kernel(
        _e_body,
        out_type=jax.ShapeDtypeStruct((_N, _D), jnp.float32),
        mesh=mesh,
        compiler_params=scp,
        scratch_types=[
            [pltpu.VMEM((_EGRP,), jnp.int32) for _ in range(4)],
            [pltpu.VMEM((_EGRP,), jnp.float32) for _ in range(4)],
            [pltpu.VMEM((_EGRP, _D), jnp.float32) for _ in range(4)],
            pltpu.SemaphoreType.DMA,
            pltpu.SemaphoreType.DMA,
        ],
    )(out_rows, pos, g1, g2)

    return y.reshape(_B, _S, _D)


def kernel(x, W_g, W_noise, W1, b1, W2, b2, k):
    return _moe(x, W_g, W_noise, W1, b1, W2, b2)
